# Initial kernel scaffold; baseline (speedup 1.0000x reference)
import jax
import jax.numpy as jnp
from jax.experimental import pallas as pl


def kernel(x_mark, minute_table, hour_table):
    raise NotImplementedError("write your pallas kernel here")


# speedup vs baseline: 4.5156x; 4.5156x over previous
import functools

import jax
import jax.numpy as jnp
from jax import lax
from jax.experimental import pallas as pl
from jax.experimental.pallas import tpu as pltpu
from jax.experimental.pallas import tpu_sc as plsc

D_MODEL = 64
MIN_ROWS = 60
HOUR_ROWS = 24
CT_ROWS = MIN_ROWS * HOUR_ROWS

NC, NS = 2, 16
NW = NC * NS

B, T = 4096, 200
N = B * T
ROWS_PER_TILE = N // NW
CHUNK = 1024
NGRP = CHUNK // 16
NSEG = CHUNK // 128
NCHUNK = ROWS_PER_TILE // CHUNK


def _ct_body(minute_ref, hour_ref, out_ref):
    m = pl.program_id(0)
    row = minute_ref[pl.ds(m, 1), :]
    out_ref[...] = hour_ref[...] + row


def _build_ct(minute_table, hour_table):
    return pl.pallas_call(
        _ct_body,
        grid=(MIN_ROWS,),
        in_specs=[
            pl.BlockSpec((MIN_ROWS, D_MODEL), lambda m: (0, 0)),
            pl.BlockSpec((HOUR_ROWS, D_MODEL), lambda m: (0, 0)),
        ],
        out_specs=pl.BlockSpec((HOUR_ROWS, D_MODEL), lambda m: (m, 0)),
        out_shape=jax.ShapeDtypeStruct((CT_ROWS, D_MODEL), jnp.float32),
    )(minute_table, hour_table)


@functools.partial(
    pl.kernel,
    out_type=jax.ShapeDtypeStruct((N, D_MODEL), jnp.float32),
    mesh=plsc.VectorSubcoreMesh(
        core_axis_name="c", subcore_axis_name="s",
        num_cores=NC, num_subcores=NS,
    ),
    scratch_types=[
        pltpu.VMEM((CHUNK * 5,), jnp.int32),
        pltpu.VMEM((NSEG, 128), jnp.int32),
        pltpu.VMEM((CHUNK, D_MODEL), jnp.float32),
        pltpu.SemaphoreType.DMA,
    ],
    compiler_params=pltpu.CompilerParams(
        needs_layout_passes=False, use_tc_tiling_on_sc=False),
)
def _sc_lookup(x_hbm, ct_hbm, out_hbm, xv, idxv, rows, gsem):
    wid = lax.axis_index("s") * NC + lax.axis_index("c")
    base0 = wid * ROWS_PER_TILE
    lane5 = lax.iota(jnp.int32, 16) * 5

    def chunk_body(k, carry):
        base = base0 + k * CHUNK
        pltpu.sync_copy(x_hbm.at[pl.ds(base * 5, CHUNK * 5)], xv)

        def grp(j, carry2):
            p = lane5 + j * 80
            m = plsc.load_gather(xv, [p + 4])
            h = plsc.load_gather(xv, [p + 3])
            idxv[j // 8, pl.ds((j % 8) * 16, 16)] = m * HOUR_ROWS + h
            return carry2

        lax.fori_loop(0, NGRP, grp, 0)

        copies = [
            pltpu.async_copy(
                ct_hbm.at[idxv.at[i]], rows.at[pl.ds(i * 128, 128)], gsem)
            for i in range(NSEG)
        ]
        for cp in copies:
            cp.wait()
        pltpu.sync_copy(rows, out_hbm.at[pl.ds(base, CHUNK)])
        return carry

    lax.fori_loop(0, NCHUNK, chunk_body, 0)


def kernel(x_mark, minute_table, hour_table):
    ct = _build_ct(minute_table, hour_table)
    x_flat = x_mark.astype(jnp.int32).reshape(N * 5)
    out = _sc_lookup(x_flat, ct)
    return out.reshape(B, T, D_MODEL)
